# baseline (speedup 1.0000x reference)
import jax
import jax.numpy as jnp
from jax.experimental import pallas as pl
from jax.experimental.pallas import tpu as pltpu

F32 = jnp.float32


def _ceil_to(a, m):
    return ((a + m - 1) // m) * m


def _dense_relu(x3, w3, b3, tb):
    two, R, K = x3.shape
    O = w3.shape[2]

    def body(x_ref, w_ref, b_ref, o_ref):
        r = jnp.dot(x_ref[0], w_ref[0], preferred_element_type=F32) + b_ref[0]
        o_ref[...] = jnp.maximum(r, 0.0)[None]

    return pl.pallas_call(
        body,
        grid=(two, R // tb),
        in_specs=[
            pl.BlockSpec((1, tb, K), lambda m, i: (m, i, 0)),
            pl.BlockSpec((1, K, O), lambda m, i: (m, 0, 0)),
            pl.BlockSpec((1, 1, O), lambda m, i: (m, 0, 0)),
        ],
        out_specs=pl.BlockSpec((1, tb, O), lambda m, i: (m, i, 0)),
        out_shape=jax.ShapeDtypeStruct((two, R, O), F32),
    )(x3, w3, b3)


def _msg(hs3, rew3, wt3, b23, teb, D):
    two, EP, DP = hs3.shape

    def body(hs_ref, rew_ref, wt_ref, b2_ref, o_ref):
        hs = hs_ref[0]
        T = jnp.dot(hs, wt_ref[0], preferred_element_type=F32)
        acc = jnp.dot(hs, b2_ref[0], preferred_element_type=F32)
        rew = rew_ref[0]
        for k in range(D):
            acc = acc + rew[:, k:k + 1] * T[:, k * DP:(k + 1) * DP]
        o_ref[...] = acc[None]

    return pl.pallas_call(
        body,
        grid=(two, EP // teb),
        in_specs=[
            pl.BlockSpec((1, teb, DP), lambda m, i: (m, i, 0)),
            pl.BlockSpec((1, teb, DP), lambda m, i: (m, i, 0)),
            pl.BlockSpec((1, DP, D * DP), lambda m, i: (m, 0, 0)),
            pl.BlockSpec((1, DP, DP), lambda m, i: (m, 0, 0)),
        ],
        out_specs=pl.BlockSpec((1, teb, DP), lambda m, i: (m, i, 0)),
        out_shape=jax.ShapeDtypeStruct((two, EP, DP), F32),
    )(hs3, rew3, wt3, b23)


def _update(p3, h3, cb3, w3, b3, tb):
    two, R, DP = p3.shape

    def body(p_ref, h_ref, cb_ref, w_ref, b_ref, o_ref):
        m = jnp.maximum(p_ref[0] + cb_ref[0] + h_ref[0], 0.0)
        r = jnp.dot(m, w_ref[0], preferred_element_type=F32) + b_ref[0]
        o_ref[...] = jnp.maximum(r, 0.0)[None]

    return pl.pallas_call(
        body,
        grid=(two, R // tb),
        in_specs=[
            pl.BlockSpec((1, tb, DP), lambda m, i: (m, i, 0)),
            pl.BlockSpec((1, tb, DP), lambda m, i: (m, i, 0)),
            pl.BlockSpec((1, 1, DP), lambda m, i: (m, 0, 0)),
            pl.BlockSpec((1, DP, DP), lambda m, i: (m, 0, 0)),
            pl.BlockSpec((1, 1, DP), lambda m, i: (m, 0, 0)),
        ],
        out_specs=pl.BlockSpec((1, tb, DP), lambda m, i: (m, i, 0)),
        out_shape=jax.ShapeDtypeStruct((two, R, DP), F32),
    )(p3, h3, cb3, w3, b3)


def _epilogue(h3, x3, gid3, fc1_w, fc1_b, fc2_w, fc2_b, fc3_w, fc3_b, tb, D, NG):
    two, NPAD, DP = h3.shape
    nb = NPAD // tb

    def body(h_ref, x_ref, gid_ref, w1_ref, b1_ref, w2_ref, b2_ref, w3_ref,
             b3_ref, o_ref, sums_ref):
        m = pl.program_id(0)
        i = pl.program_id(1)

        @pl.when(i == 0)
        def _():
            sums_ref[pl.ds(m * NG, NG), :] = jnp.zeros((NG, DP), F32)

        f = h_ref[0] + x_ref[0]
        col = jax.lax.broadcasted_iota(jnp.int32, (tb, DP), 1)
        f = jnp.where(col == DP - 1, 1.0, f)
        gid = gid_ref[0]
        oh = (jax.lax.broadcasted_iota(jnp.int32, (NG, tb), 0) == gid).astype(F32)
        sums_ref[pl.ds(m * NG, NG), :] += jnp.dot(oh, f, preferred_element_type=F32)

        @pl.when((m == 1) & (i == nb - 1))
        def _():
            s0 = sums_ref[pl.ds(0, NG), :]
            s1 = sums_ref[pl.ds(NG, NG), :]
            m0 = s0[:, :D] / jnp.maximum(s0[:, DP - 1:DP], 1.0)
            m1 = s1[:, :D] / jnp.maximum(s1[:, DP - 1:DP], 1.0)
            g0 = jnp.sum(m0, axis=0, keepdims=True) / NG
            g1 = jnp.sum(m1, axis=0, keepdims=True) / NG
            comb = jnp.concatenate([g0, g0, g1, g1], axis=1)
            h1 = jnp.maximum(
                jnp.dot(comb, w1_ref[...], preferred_element_type=F32) + b1_ref[...], 0.0)
            h2 = jnp.maximum(
                jnp.dot(h1, w2_ref[...], preferred_element_type=F32) + b2_ref[...], 0.0)
            o_ref[...] = jnp.dot(h2, w3_ref[...], preferred_element_type=F32) + b3_ref[...]

    return pl.pallas_call(
        body,
        grid=(two, nb),
        in_specs=[
            pl.BlockSpec((1, tb, DP), lambda m, i: (m, i, 0)),
            pl.BlockSpec((1, tb, DP), lambda m, i: (m, i, 0)),
            pl.BlockSpec((1, 1, tb), lambda m, i: (m * nb + i, 0, 0)),
            pl.BlockSpec(fc1_w.shape, lambda m, i: (0, 0)),
            pl.BlockSpec(fc1_b.shape, lambda m, i: (0, 0)),
            pl.BlockSpec(fc2_w.shape, lambda m, i: (0, 0)),
            pl.BlockSpec(fc2_b.shape, lambda m, i: (0, 0)),
            pl.BlockSpec(fc3_w.shape, lambda m, i: (0, 0)),
            pl.BlockSpec(fc3_b.shape, lambda m, i: (0, 0)),
        ],
        out_specs=pl.BlockSpec((1, 1), lambda m, i: (0, 0)),
        out_shape=jax.ShapeDtypeStruct((1, 1), F32),
        scratch_shapes=[pltpu.VMEM((2 * NG, DP), F32)],
    )(h3, x3, gid3, fc1_w, fc1_b, fc2_w, fc2_b, fc3_w, fc3_b)


def _pad2(a, r, c):
    return jnp.pad(a, ((0, r - a.shape[0]), (0, c - a.shape[1])))


def kernel(solute_x, solute_edge_index, solute_w, solute_graph_ids,
           solvent_x, solvent_edge_index, solvent_w, solvent_graph_ids,
           su_lin0_w, su_lin0_b, su_en_w1, su_en_b1, su_en_w2, su_en_b2,
           su_conv_b, su_msg_w, su_msg_b,
           sv_lin0_w, sv_lin0_b, sv_en_w1, sv_en_b1, sv_en_w2, sv_en_b2,
           sv_conv_b, sv_msg_w, sv_msg_b,
           fc1_w, fc1_b, fc2_w, fc2_b, fc3_w, fc3_b):
    N, D = solute_x.shape
    E, DE = solute_w.shape
    NG = 256
    DP = 48
    DEP = 16
    TB = 512
    TEB = 256
    NP = _ceil_to(N + 1, TB)
    EP = _ceil_to(E, TB)

    x3 = jnp.stack([_pad2(solute_x, NP, DP), _pad2(solvent_x, NP, DP)])
    w3 = jnp.stack([_pad2(solute_w, EP, DEP), _pad2(solvent_w, EP, DEP)])

    lin0_w3 = jnp.stack([_pad2(su_lin0_w, DP, DP), _pad2(sv_lin0_w, DP, DP)])
    lin0_b3 = jnp.stack([_pad2(su_lin0_b[None], 1, DP), _pad2(sv_lin0_b[None], 1, DP)])
    en_w13 = jnp.stack([_pad2(su_en_w1, DEP, DP), _pad2(sv_en_w1, DEP, DP)])
    en_b13 = jnp.stack([_pad2(su_en_b1[None], 1, DP), _pad2(sv_en_b1[None], 1, DP)])

    def _wt(en_w2):
        w2 = en_w2.reshape(D, D, D)
        wt = jnp.transpose(w2, (1, 0, 2))
        wt = jnp.pad(wt, ((0, DP - D), (0, 0), (0, DP - D)))
        return wt.reshape(DP, D * DP)

    wt3 = jnp.stack([_wt(su_en_w2), _wt(sv_en_w2)])
    b23 = jnp.stack([_pad2(su_en_b2.reshape(D, D), DP, DP),
                     _pad2(sv_en_b2.reshape(D, D), DP, DP)])
    cb3 = jnp.stack([_pad2(su_conv_b[None], 1, DP), _pad2(sv_conv_b[None], 1, DP)])
    msg_w3 = jnp.stack([_pad2(su_msg_w, DP, DP), _pad2(sv_msg_w, DP, DP)])
    msg_b3 = jnp.stack([_pad2(su_msg_b[None], 1, DP), _pad2(sv_msg_b[None], 1, DP)])

    def _pad_idx(idx, fill):
        return jnp.concatenate(
            [idx.astype(jnp.int32), jnp.full((EP - E,), fill, jnp.int32)])

    src_all = jnp.concatenate([
        _pad_idx(solute_edge_index[0], N),
        _pad_idx(solvent_edge_index[0], N) + NP])
    dst_all = jnp.concatenate([
        _pad_idx(solute_edge_index[1], N),
        _pad_idx(solvent_edge_index[1], N) + NP])

    def _pad_gid(g):
        return jnp.concatenate(
            [g.astype(jnp.int32), jnp.full((NP - N,), NG, jnp.int32)])

    nbg = NP // TB
    gid3 = jnp.concatenate([_pad_gid(solute_graph_ids),
                            _pad_gid(solvent_graph_ids)]).reshape(2 * nbg, 1, TB)

    h3 = _dense_relu(x3, lin0_w3, lin0_b3, TB)
    rew3 = _dense_relu(w3, en_w13, en_b13, TB)

    for _ in range(6):
        htab = h3.reshape(2 * NP, DP)
        hs = jnp.take(htab, src_all, axis=0).reshape(2, EP, DP)
        msg = _msg(hs, rew3, wt3, b23, TEB, D)
        p = jax.ops.segment_sum(msg.reshape(2 * EP, DP), dst_all,
                                num_segments=2 * NP).reshape(2, NP, DP)
        h3 = _update(p, h3, cb3, msg_w3, msg_b3, TB)

    return _epilogue(h3, x3, gid3, fc1_w, fc1_b[None], fc2_w, fc2_b[None],
                     fc3_w, fc3_b[None], TB, D, NG)


# speedup vs baseline: 2.9006x; 2.9006x over previous
import functools

import jax
import jax.numpy as jnp
from jax import lax
from jax.experimental import pallas as pl
from jax.experimental.pallas import tpu as pltpu
from jax.experimental.pallas import tpu_sc as plsc

F32 = jnp.float32
NC, NS = 2, 16
CHJ = 128


def _make_gather(dp, epp):
    per_tile = epp // NS
    nj = per_tile // CHJ
    mesh = plsc.VectorSubcoreMesh(core_axis_name="c", subcore_axis_name="s")

    def gather_k(table_hbm, idx_hbm, out_hbm, idx_v, rows_v, sem):
        c = lax.axis_index("c")
        s = lax.axis_index("s")
        w = c * NS + s
        pltpu.sync_copy(idx_hbm.at[w], idx_v)

        def body(j, carry):
            pltpu.async_copy(table_hbm.at[idx_v.at[j]], rows_v, sem).wait()
            pltpu.sync_copy(rows_v, out_hbm.at[pl.ds(w * per_tile + j * CHJ, CHJ)])
            return carry

        lax.fori_loop(0, nj, body, 0)

    return functools.partial(
        pl.kernel, mesh=mesh,
        out_type=jax.ShapeDtypeStruct((2 * epp, dp), F32),
        compiler_params=pltpu.CompilerParams(use_tc_tiling_on_sc=False),
        scratch_types=[
            pltpu.VMEM((nj, CHJ), jnp.int32),
            pltpu.VMEM((CHJ, dp), F32),
            pltpu.SemaphoreType.DMA,
        ],
    )(gather_k)


def _make_scatter(np_, dp, epp):
    per_tile = epp // NS
    nj = per_tile // CHJ
    rp = np_ // NS
    mesh = plsc.VectorSubcoreMesh(core_axis_name="c", subcore_axis_name="s")

    def scatter_k(msg_hbm, dst_hbm, zeros_hbm, out_hbm, idx_v, rows_v, acc_sh):
        c = lax.axis_index("c")
        s = lax.axis_index("s")
        w = c * NS + s
        pltpu.sync_copy(zeros_hbm.at[pl.ds(s * rp, rp)],
                        acc_sh.at[pl.ds(s * rp, rp)])
        plsc.subcore_barrier()
        pltpu.sync_copy(dst_hbm.at[w], idx_v)

        for j in range(nj):
            pltpu.sync_copy(msg_hbm.at[pl.ds(w * per_tile + j * CHJ, CHJ)], rows_v)
            pltpu.sync_copy(rows_v, acc_sh.at[idx_v.at[j]], add=True)
        plsc.subcore_barrier()
        pltpu.sync_copy(acc_sh.at[pl.ds(s * rp, rp)],
                        out_hbm.at[pl.ds(c * np_ + s * rp, rp)])

    return functools.partial(
        pl.kernel, mesh=mesh,
        out_type=jax.ShapeDtypeStruct((2 * np_, dp), F32),
        compiler_params=pltpu.CompilerParams(use_tc_tiling_on_sc=False),
        scratch_types=[
            pltpu.VMEM((nj, CHJ), jnp.int32),
            pltpu.VMEM((CHJ, dp), F32),
            pltpu.VMEM_SHARED((np_, dp), F32),
        ],
    )(scatter_k)


def _ceil_to(a, m):
    return ((a + m - 1) // m) * m


def _dense_relu(x3, w3, b3, tb):
    two, R, K = x3.shape
    O = w3.shape[2]

    def body(x_ref, w_ref, b_ref, o_ref):
        r = jnp.dot(x_ref[0], w_ref[0], preferred_element_type=F32) + b_ref[0]
        o_ref[...] = jnp.maximum(r, 0.0)[None]

    return pl.pallas_call(
        body,
        grid=(two, R // tb),
        in_specs=[
            pl.BlockSpec((1, tb, K), lambda m, i: (m, i, 0)),
            pl.BlockSpec((1, K, O), lambda m, i: (m, 0, 0)),
            pl.BlockSpec((1, 1, O), lambda m, i: (m, 0, 0)),
        ],
        out_specs=pl.BlockSpec((1, tb, O), lambda m, i: (m, i, 0)),
        out_shape=jax.ShapeDtypeStruct((two, R, O), F32),
    )(x3, w3, b3)


def _msg(hs3, rew3, wt3, rep, sel, b23, teb, D):
    two, EP, DP = hs3.shape
    KO = D * DP

    def body(hs_ref, rew_ref, wt_ref, rep_ref, sel_ref, b2_ref, o_ref):
        hs = hs_ref[0]
        T = jnp.dot(hs, wt_ref[0], preferred_element_type=F32)
        Rx = jnp.dot(rew_ref[0], rep_ref[...], preferred_element_type=F32)
        acc = jnp.dot(T * Rx, sel_ref[...], preferred_element_type=F32)
        acc = acc + jnp.dot(hs, b2_ref[0], preferred_element_type=F32)
        o_ref[...] = acc[None]

    return pl.pallas_call(
        body,
        grid=(two, EP // teb),
        in_specs=[
            pl.BlockSpec((1, teb, DP), lambda m, i: (m, i, 0)),
            pl.BlockSpec((1, teb, DP), lambda m, i: (m, i, 0)),
            pl.BlockSpec((1, DP, KO), lambda m, i: (m, 0, 0)),
            pl.BlockSpec((DP, KO), lambda m, i: (0, 0)),
            pl.BlockSpec((KO, DP), lambda m, i: (0, 0)),
            pl.BlockSpec((1, DP, DP), lambda m, i: (m, 0, 0)),
        ],
        out_specs=pl.BlockSpec((1, teb, DP), lambda m, i: (m, i, 0)),
        out_shape=jax.ShapeDtypeStruct((two, EP, DP), F32),
    )(hs3, rew3, wt3, rep, sel, b23)


def _update(p3, h3, cb3, w3, b3, tb):
    two, R, DP = p3.shape

    def body(p_ref, h_ref, cb_ref, w_ref, b_ref, o_ref):
        m = jnp.maximum(p_ref[0] + cb_ref[0] + h_ref[0], 0.0)
        r = jnp.dot(m, w_ref[0], preferred_element_type=F32) + b_ref[0]
        o_ref[...] = jnp.maximum(r, 0.0)[None]

    return pl.pallas_call(
        body,
        grid=(two, R // tb),
        in_specs=[
            pl.BlockSpec((1, tb, DP), lambda m, i: (m, i, 0)),
            pl.BlockSpec((1, tb, DP), lambda m, i: (m, i, 0)),
            pl.BlockSpec((1, 1, DP), lambda m, i: (m, 0, 0)),
            pl.BlockSpec((1, DP, DP), lambda m, i: (m, 0, 0)),
            pl.BlockSpec((1, 1, DP), lambda m, i: (m, 0, 0)),
        ],
        out_specs=pl.BlockSpec((1, tb, DP), lambda m, i: (m, i, 0)),
        out_shape=jax.ShapeDtypeStruct((two, R, DP), F32),
    )(p3, h3, cb3, w3, b3)


def _epilogue(h3, x3, gid3, fc1_w, fc1_b, fc2_w, fc2_b, fc3_w, fc3_b, tb, D, NG):
    two, NPAD, DP = h3.shape
    nb = NPAD // tb

    def body(h_ref, x_ref, gid_ref, w1_ref, b1_ref, w2_ref, b2_ref, w3_ref,
             b3_ref, o_ref, sums_ref):
        m = pl.program_id(0)
        i = pl.program_id(1)

        @pl.when(i == 0)
        def _():
            sums_ref[pl.ds(m * NG, NG), :] = jnp.zeros((NG, DP), F32)

        f = h_ref[0] + x_ref[0]
        col = jax.lax.broadcasted_iota(jnp.int32, (tb, DP), 1)
        f = jnp.where(col == DP - 1, 1.0, f)
        gid = gid_ref[0]
        oh = (jax.lax.broadcasted_iota(jnp.int32, (NG, tb), 0) == gid).astype(F32)
        sums_ref[pl.ds(m * NG, NG), :] += jnp.dot(oh, f, preferred_element_type=F32)

        @pl.when((m == 1) & (i == nb - 1))
        def _():
            s0 = sums_ref[pl.ds(0, NG), :]
            s1 = sums_ref[pl.ds(NG, NG), :]
            m0 = s0[:, :D] / jnp.maximum(s0[:, DP - 1:DP], 1.0)
            m1 = s1[:, :D] / jnp.maximum(s1[:, DP - 1:DP], 1.0)
            g0 = jnp.sum(m0, axis=0, keepdims=True) / NG
            g1 = jnp.sum(m1, axis=0, keepdims=True) / NG
            comb = jnp.concatenate([g0, g0, g1, g1], axis=1)
            h1 = jnp.maximum(
                jnp.dot(comb, w1_ref[...], preferred_element_type=F32) + b1_ref[...], 0.0)
            h2 = jnp.maximum(
                jnp.dot(h1, w2_ref[...], preferred_element_type=F32) + b2_ref[...], 0.0)
            o_ref[...] = jnp.dot(h2, w3_ref[...], preferred_element_type=F32) + b3_ref[...]

    return pl.pallas_call(
        body,
        grid=(two, nb),
        in_specs=[
            pl.BlockSpec((1, tb, DP), lambda m, i: (m, i, 0)),
            pl.BlockSpec((1, tb, DP), lambda m, i: (m, i, 0)),
            pl.BlockSpec((1, 1, tb), lambda m, i: (m * nb + i, 0, 0)),
            pl.BlockSpec(fc1_w.shape, lambda m, i: (0, 0)),
            pl.BlockSpec(fc1_b.shape, lambda m, i: (0, 0)),
            pl.BlockSpec(fc2_w.shape, lambda m, i: (0, 0)),
            pl.BlockSpec(fc2_b.shape, lambda m, i: (0, 0)),
            pl.BlockSpec(fc3_w.shape, lambda m, i: (0, 0)),
            pl.BlockSpec(fc3_b.shape, lambda m, i: (0, 0)),
        ],
        out_specs=pl.BlockSpec((1, 1), lambda m, i: (0, 0)),
        out_shape=jax.ShapeDtypeStruct((1, 1), F32),
        scratch_shapes=[pltpu.VMEM((2 * NG, DP), F32)],
    )(h3, x3, gid3, fc1_w, fc1_b, fc2_w, fc2_b, fc3_w, fc3_b)


def _pad2(a, r, c):
    return jnp.pad(a, ((0, r - a.shape[0]), (0, c - a.shape[1])))


def kernel(solute_x, solute_edge_index, solute_w, solute_graph_ids,
           solvent_x, solvent_edge_index, solvent_w, solvent_graph_ids,
           su_lin0_w, su_lin0_b, su_en_w1, su_en_b1, su_en_w2, su_en_b2,
           su_conv_b, su_msg_w, su_msg_b,
           sv_lin0_w, sv_lin0_b, sv_en_w1, sv_en_b1, sv_en_w2, sv_en_b2,
           sv_conv_b, sv_msg_w, sv_msg_b,
           fc1_w, fc1_b, fc2_w, fc2_b, fc3_w, fc3_b):
    N, D = solute_x.shape
    E, DE = solute_w.shape
    NG = 256
    DP = 48
    DEP = 16
    TB = 512
    TEB = 256
    NP = _ceil_to(N + 1, TB)
    EP = _ceil_to(E, NS * CHJ)
    NJ = EP // NS // CHJ

    x3 = jnp.stack([_pad2(solute_x, NP, DP), _pad2(solvent_x, NP, DP)])
    w3 = jnp.stack([_pad2(solute_w, EP, DEP), _pad2(solvent_w, EP, DEP)])

    lin0_w3 = jnp.stack([_pad2(su_lin0_w, DP, DP), _pad2(sv_lin0_w, DP, DP)])
    lin0_b3 = jnp.stack([_pad2(su_lin0_b[None], 1, DP), _pad2(sv_lin0_b[None], 1, DP)])
    en_w13 = jnp.stack([_pad2(su_en_w1, DEP, DP), _pad2(sv_en_w1, DEP, DP)])
    en_b13 = jnp.stack([_pad2(su_en_b1[None], 1, DP), _pad2(sv_en_b1[None], 1, DP)])

    def _wt(en_w2):
        w2 = en_w2.reshape(D, D, D)
        wt = jnp.transpose(w2, (1, 0, 2))
        wt = jnp.pad(wt, ((0, DP - D), (0, 0), (0, DP - D)))
        return wt.reshape(DP, D * DP)

    wt3 = jnp.stack([_wt(su_en_w2), _wt(sv_en_w2)])
    KO = D * DP
    rep = (jnp.arange(KO)[None, :] // DP
           == jnp.arange(DP)[:, None]).astype(F32)
    sel = (jnp.arange(KO)[:, None] % DP
           == jnp.arange(DP)[None, :]).astype(F32)
    b23 = jnp.stack([_pad2(su_en_b2.reshape(D, D), DP, DP),
                     _pad2(sv_en_b2.reshape(D, D), DP, DP)])
    cb3 = jnp.stack([_pad2(su_conv_b[None], 1, DP), _pad2(sv_conv_b[None], 1, DP)])
    msg_w3 = jnp.stack([_pad2(su_msg_w, DP, DP), _pad2(sv_msg_w, DP, DP)])
    msg_b3 = jnp.stack([_pad2(su_msg_b[None], 1, DP), _pad2(sv_msg_b[None], 1, DP)])

    def _pad_idx(idx, fill):
        return jnp.concatenate(
            [idx.astype(jnp.int32), jnp.full((EP - E,), fill, jnp.int32)])

    src3 = jnp.concatenate([
        _pad_idx(solute_edge_index[0], N),
        _pad_idx(solvent_edge_index[0], N) + NP]).reshape(2 * NS, NJ, CHJ)
    dst3 = jnp.concatenate([
        _pad_idx(solute_edge_index[1], N),
        _pad_idx(solvent_edge_index[1], N)]).reshape(2 * NS, NJ, CHJ)
    zeros_np = jnp.zeros((NP, DP), F32)

    def _pad_gid(g):
        return jnp.concatenate(
            [g.astype(jnp.int32), jnp.full((NP - N,), NG, jnp.int32)])

    nbg = NP // TB
    gid3 = jnp.concatenate([_pad_gid(solute_graph_ids),
                            _pad_gid(solvent_graph_ids)]).reshape(2 * nbg, 1, TB)

    h3 = _dense_relu(x3, lin0_w3, lin0_b3, TB)
    rew3 = _dense_relu(w3, en_w13, en_b13, TB)

    gather_k = _make_gather(DP, EP)
    scatter_k = _make_scatter(NP, DP, EP)
    for _ in range(6):
        hs = gather_k(h3.reshape(2 * NP, DP), src3).reshape(2, EP, DP)
        msg = _msg(hs, rew3, wt3, rep, sel, b23, TEB, D)
        p = scatter_k(msg.reshape(2 * EP, DP), dst3,
                      zeros_np).reshape(2, NP, DP)
        h3 = _update(p, h3, cb3, msg_w3, msg_b3, TB)

    return _epilogue(h3, x3, gid3, fc1_w, fc1_b[None], fc2_w, fc2_b[None],
                     fc3_w, fc3_b[None], TB, D, NG)


# speedup vs baseline: 3.4212x; 1.1795x over previous
import functools

import jax
import jax.numpy as jnp
from jax import lax
from jax.experimental import pallas as pl
from jax.experimental.pallas import tpu as pltpu
from jax.experimental.pallas import tpu_sc as plsc

F32 = jnp.float32
NC, NS = 2, 16
CHJ = 128


def _make_gather(dp, epp):
    per_tile = epp // NS
    nj = per_tile // CHJ
    mesh = plsc.VectorSubcoreMesh(core_axis_name="c", subcore_axis_name="s")

    def gather_k(table_hbm, idx_hbm, out_hbm, idx_v, rows_v, sem):
        c = lax.axis_index("c")
        s = lax.axis_index("s")
        w = c * NS + s
        pltpu.sync_copy(idx_hbm.at[w], idx_v)

        def body(j, carry):
            pltpu.async_copy(table_hbm.at[idx_v.at[j]], rows_v, sem).wait()
            pltpu.sync_copy(rows_v, out_hbm.at[pl.ds(w * per_tile + j * CHJ, CHJ)])
            return carry

        lax.fori_loop(0, nj, body, 0)

    return functools.partial(
        pl.kernel, mesh=mesh,
        out_type=jax.ShapeDtypeStruct((2 * epp, dp), F32),
        compiler_params=pltpu.CompilerParams(use_tc_tiling_on_sc=False),
        scratch_types=[
            pltpu.VMEM((nj, CHJ), jnp.int32),
            pltpu.VMEM((CHJ, dp), F32),
            pltpu.SemaphoreType.DMA,
        ],
    )(gather_k)


def _make_scatter(np_, dp, epp):
    per_tile = epp // NS
    nj = per_tile // CHJ
    rp = np_ // NS
    mesh = plsc.VectorSubcoreMesh(core_axis_name="c", subcore_axis_name="s")

    def scatter_k(msg_hbm, dst_hbm, zeros_hbm, out_hbm, idx_v, rows_v, acc_sh):
        c = lax.axis_index("c")
        s = lax.axis_index("s")
        w = c * NS + s
        pltpu.sync_copy(zeros_hbm.at[pl.ds(s * rp, rp)],
                        acc_sh.at[pl.ds(s * rp, rp)])
        plsc.subcore_barrier()
        pltpu.sync_copy(dst_hbm.at[w], idx_v)

        for j in range(nj):
            pltpu.sync_copy(msg_hbm.at[pl.ds(w * per_tile + j * CHJ, CHJ)], rows_v)
            pltpu.sync_copy(rows_v, acc_sh.at[idx_v.at[j]], add=True)
        plsc.subcore_barrier()
        pltpu.sync_copy(acc_sh.at[pl.ds(s * rp, rp)],
                        out_hbm.at[pl.ds(c * np_ + s * rp, rp)])

    return functools.partial(
        pl.kernel, mesh=mesh,
        out_type=jax.ShapeDtypeStruct((2 * np_, dp), F32),
        compiler_params=pltpu.CompilerParams(use_tc_tiling_on_sc=False),
        scratch_types=[
            pltpu.VMEM((nj, CHJ), jnp.int32),
            pltpu.VMEM((CHJ, dp), F32),
            pltpu.VMEM_SHARED((np_, dp), F32),
        ],
    )(scatter_k)


def _ceil_to(a, m):
    return ((a + m - 1) // m) * m


def _dense_relu(x3, w3, b3, tb):
    two, R, K = x3.shape
    O = w3.shape[2]

    def body(x_ref, w_ref, b_ref, o_ref):
        r = jnp.dot(x_ref[0], w_ref[0], preferred_element_type=F32) + b_ref[0]
        o_ref[...] = jnp.maximum(r, 0.0)[None]

    return pl.pallas_call(
        body,
        grid=(two, R // tb),
        in_specs=[
            pl.BlockSpec((1, tb, K), lambda m, i: (m, i, 0)),
            pl.BlockSpec((1, K, O), lambda m, i: (m, 0, 0)),
            pl.BlockSpec((1, 1, O), lambda m, i: (m, 0, 0)),
        ],
        out_specs=pl.BlockSpec((1, tb, O), lambda m, i: (m, i, 0)),
        out_shape=jax.ShapeDtypeStruct((two, R, O), F32),
    )(x3, w3, b3)


def _msg(hs3, rew3, wt3, rep, sel, b23, teb, D):
    two, EP, DP = hs3.shape
    KO = D * D

    def body(hs_ref, rew_ref, wt_ref, rep_ref, sel_ref, b2_ref, o_ref):
        hs = hs_ref[0]
        T = jnp.dot(hs, wt_ref[0], preferred_element_type=F32)
        Rx = jnp.dot(rew_ref[0], rep_ref[...], preferred_element_type=F32)
        acc = jnp.dot(T * Rx, sel_ref[...], preferred_element_type=F32)
        acc = acc + jnp.dot(hs, b2_ref[0], preferred_element_type=F32)
        o_ref[...] = acc[None]

    return pl.pallas_call(
        body,
        grid=(two, EP // teb),
        in_specs=[
            pl.BlockSpec((1, teb, DP), lambda m, i: (m, i, 0)),
            pl.BlockSpec((1, teb, DP), lambda m, i: (m, i, 0)),
            pl.BlockSpec((1, DP, D * D), lambda m, i: (m, 0, 0)),
            pl.BlockSpec((DP, D * D), lambda m, i: (0, 0)),
            pl.BlockSpec((D * D, DP), lambda m, i: (0, 0)),
            pl.BlockSpec((1, DP, DP), lambda m, i: (m, 0, 0)),
        ],
        out_specs=pl.BlockSpec((1, teb, DP), lambda m, i: (m, i, 0)),
        out_shape=jax.ShapeDtypeStruct((two, EP, DP), F32),
    )(hs3, rew3, wt3, rep, sel, b23)


def _update(p3, h3, cb3, w3, b3, tb):
    two, R, DP = p3.shape

    def body(p_ref, h_ref, cb_ref, w_ref, b_ref, o_ref):
        m = jnp.maximum(p_ref[0] + cb_ref[0] + h_ref[0], 0.0)
        r = jnp.dot(m, w_ref[0], preferred_element_type=F32) + b_ref[0]
        o_ref[...] = jnp.maximum(r, 0.0)[None]

    return pl.pallas_call(
        body,
        grid=(two, R // tb),
        in_specs=[
            pl.BlockSpec((1, tb, DP), lambda m, i: (m, i, 0)),
            pl.BlockSpec((1, tb, DP), lambda m, i: (m, i, 0)),
            pl.BlockSpec((1, 1, DP), lambda m, i: (m, 0, 0)),
            pl.BlockSpec((1, DP, DP), lambda m, i: (m, 0, 0)),
            pl.BlockSpec((1, 1, DP), lambda m, i: (m, 0, 0)),
        ],
        out_specs=pl.BlockSpec((1, tb, DP), lambda m, i: (m, i, 0)),
        out_shape=jax.ShapeDtypeStruct((two, R, DP), F32),
    )(p3, h3, cb3, w3, b3)


def _epilogue(h3, x3, gid3, fc1_w, fc1_b, fc2_w, fc2_b, fc3_w, fc3_b, tb, D, NG):
    two, NPAD, DP = h3.shape
    nb = NPAD // tb

    def body(h_ref, x_ref, gid_ref, w1_ref, b1_ref, w2_ref, b2_ref, w3_ref,
             b3_ref, o_ref, sums_ref):
        m = pl.program_id(0)
        i = pl.program_id(1)

        @pl.when(i == 0)
        def _():
            sums_ref[pl.ds(m * NG, NG), :] = jnp.zeros((NG, DP), F32)

        f = h_ref[0] + x_ref[0]
        col = jax.lax.broadcasted_iota(jnp.int32, (tb, DP), 1)
        f = jnp.where(col == DP - 1, 1.0, f)
        gid = gid_ref[0]
        oh = (jax.lax.broadcasted_iota(jnp.int32, (NG, tb), 0) == gid).astype(F32)
        sums_ref[pl.ds(m * NG, NG), :] += jnp.dot(oh, f, preferred_element_type=F32)

        @pl.when((m == 1) & (i == nb - 1))
        def _():
            s0 = sums_ref[pl.ds(0, NG), :]
            s1 = sums_ref[pl.ds(NG, NG), :]
            m0 = s0[:, :D] / jnp.maximum(s0[:, DP - 1:DP], 1.0)
            m1 = s1[:, :D] / jnp.maximum(s1[:, DP - 1:DP], 1.0)
            g0 = jnp.sum(m0, axis=0, keepdims=True) / NG
            g1 = jnp.sum(m1, axis=0, keepdims=True) / NG
            comb = jnp.concatenate([g0, g0, g1, g1], axis=1)
            h1 = jnp.maximum(
                jnp.dot(comb, w1_ref[...], preferred_element_type=F32) + b1_ref[...], 0.0)
            h2 = jnp.maximum(
                jnp.dot(h1, w2_ref[...], preferred_element_type=F32) + b2_ref[...], 0.0)
            o_ref[...] = jnp.dot(h2, w3_ref[...], preferred_element_type=F32) + b3_ref[...]

    return pl.pallas_call(
        body,
        grid=(two, nb),
        in_specs=[
            pl.BlockSpec((1, tb, DP), lambda m, i: (m, i, 0)),
            pl.BlockSpec((1, tb, DP), lambda m, i: (m, i, 0)),
            pl.BlockSpec((1, 1, tb), lambda m, i: (m * nb + i, 0, 0)),
            pl.BlockSpec(fc1_w.shape, lambda m, i: (0, 0)),
            pl.BlockSpec(fc1_b.shape, lambda m, i: (0, 0)),
            pl.BlockSpec(fc2_w.shape, lambda m, i: (0, 0)),
            pl.BlockSpec(fc2_b.shape, lambda m, i: (0, 0)),
            pl.BlockSpec(fc3_w.shape, lambda m, i: (0, 0)),
            pl.BlockSpec(fc3_b.shape, lambda m, i: (0, 0)),
        ],
        out_specs=pl.BlockSpec((1, 1), lambda m, i: (0, 0)),
        out_shape=jax.ShapeDtypeStruct((1, 1), F32),
        scratch_shapes=[pltpu.VMEM((2 * NG, DP), F32)],
    )(h3, x3, gid3, fc1_w, fc1_b, fc2_w, fc2_b, fc3_w, fc3_b)


def _pad2(a, r, c):
    return jnp.pad(a, ((0, r - a.shape[0]), (0, c - a.shape[1])))


def kernel(solute_x, solute_edge_index, solute_w, solute_graph_ids,
           solvent_x, solvent_edge_index, solvent_w, solvent_graph_ids,
           su_lin0_w, su_lin0_b, su_en_w1, su_en_b1, su_en_w2, su_en_b2,
           su_conv_b, su_msg_w, su_msg_b,
           sv_lin0_w, sv_lin0_b, sv_en_w1, sv_en_b1, sv_en_w2, sv_en_b2,
           sv_conv_b, sv_msg_w, sv_msg_b,
           fc1_w, fc1_b, fc2_w, fc2_b, fc3_w, fc3_b):
    N, D = solute_x.shape
    E, DE = solute_w.shape
    NG = 256
    DP = 48
    DEP = 16
    TB = 512
    TEB = 512
    NP = _ceil_to(N + 1, TB)
    EP = _ceil_to(E, NS * CHJ)
    NJ = EP // NS // CHJ

    x3 = jnp.stack([_pad2(solute_x, NP, DP), _pad2(solvent_x, NP, DP)])
    w3 = jnp.stack([_pad2(solute_w, EP, DEP), _pad2(solvent_w, EP, DEP)])

    lin0_w3 = jnp.stack([_pad2(su_lin0_w, DP, DP), _pad2(sv_lin0_w, DP, DP)])
    lin0_b3 = jnp.stack([_pad2(su_lin0_b[None], 1, DP), _pad2(sv_lin0_b[None], 1, DP)])
    en_w13 = jnp.stack([_pad2(su_en_w1, DEP, DP), _pad2(sv_en_w1, DEP, DP)])
    en_b13 = jnp.stack([_pad2(su_en_b1[None], 1, DP), _pad2(sv_en_b1[None], 1, DP)])

    def _wt(en_w2):
        w2 = en_w2.reshape(D, D, D)
        wt = jnp.transpose(w2, (1, 0, 2))
        wt = jnp.pad(wt, ((0, DP - D), (0, 0), (0, 0)))
        return wt.reshape(DP, D * D)

    wt3 = jnp.stack([_wt(su_en_w2), _wt(sv_en_w2)])
    KO = D * D
    rep = (jnp.arange(KO)[None, :] // D
           == jnp.arange(DP)[:, None]).astype(F32)
    sel = (jnp.arange(KO)[:, None] % D
           == jnp.arange(DP)[None, :]).astype(F32)
    b23 = jnp.stack([_pad2(su_en_b2.reshape(D, D), DP, DP),
                     _pad2(sv_en_b2.reshape(D, D), DP, DP)])
    cb3 = jnp.stack([_pad2(su_conv_b[None], 1, DP), _pad2(sv_conv_b[None], 1, DP)])
    msg_w3 = jnp.stack([_pad2(su_msg_w, DP, DP), _pad2(sv_msg_w, DP, DP)])
    msg_b3 = jnp.stack([_pad2(su_msg_b[None], 1, DP), _pad2(sv_msg_b[None], 1, DP)])

    def _pad_idx(idx, fill):
        return jnp.concatenate(
            [idx.astype(jnp.int32), jnp.full((EP - E,), fill, jnp.int32)])

    src3 = jnp.concatenate([
        _pad_idx(solute_edge_index[0], N),
        _pad_idx(solvent_edge_index[0], N) + NP]).reshape(2 * NS, NJ, CHJ)
    dst3 = jnp.concatenate([
        _pad_idx(solute_edge_index[1], N),
        _pad_idx(solvent_edge_index[1], N)]).reshape(2 * NS, NJ, CHJ)
    zeros_np = jnp.zeros((NP, DP), F32)

    def _pad_gid(g):
        return jnp.concatenate(
            [g.astype(jnp.int32), jnp.full((NP - N,), NG, jnp.int32)])

    nbg = NP // TB
    gid3 = jnp.concatenate([_pad_gid(solute_graph_ids),
                            _pad_gid(solvent_graph_ids)]).reshape(2 * nbg, 1, TB)

    h3 = _dense_relu(x3, lin0_w3, lin0_b3, TB)
    rew3 = _dense_relu(w3, en_w13, en_b13, TB)

    gather_k = _make_gather(DP, EP)
    scatter_k = _make_scatter(NP, DP, EP)
    for _ in range(6):
        hs = gather_k(h3.reshape(2 * NP, DP), src3).reshape(2, EP, DP)
        msg = _msg(hs, rew3, wt3, rep, sel, b23, TEB, D)
        p = scatter_k(msg.reshape(2 * EP, DP), dst3,
                      zeros_np).reshape(2, NP, DP)
        h3 = _update(p, h3, cb3, msg_w3, msg_b3, TB)

    return _epilogue(h3, x3, gid3, fc1_w, fc1_b[None], fc2_w, fc2_b[None],
                     fc3_w, fc3_b[None], TB, D, NG)


# speedup vs baseline: 3.5732x; 1.0444x over previous
import functools

import jax
import jax.numpy as jnp
from jax import lax
from jax.experimental import pallas as pl
from jax.experimental.pallas import tpu as pltpu
from jax.experimental.pallas import tpu_sc as plsc

F32 = jnp.float32
NC, NS = 2, 16
CHJ = 128


def _make_gather(dp, epp):
    per_tile = epp // NS
    nj = per_tile // CHJ
    mesh = plsc.VectorSubcoreMesh(core_axis_name="c", subcore_axis_name="s")

    def gather_k(table_hbm, idx_hbm, out_hbm, idx_v, rows_v, sem):
        c = lax.axis_index("c")
        s = lax.axis_index("s")
        w = c * NS + s
        pltpu.sync_copy(idx_hbm.at[w], idx_v)

        def body(j, carry):
            pltpu.async_copy(table_hbm.at[idx_v.at[j]], rows_v, sem).wait()
            pltpu.sync_copy(rows_v, out_hbm.at[pl.ds(w * per_tile + j * CHJ, CHJ)])
            return carry

        lax.fori_loop(0, nj, body, 0)

    return functools.partial(
        pl.kernel, mesh=mesh,
        out_type=jax.ShapeDtypeStruct((2 * epp, dp), F32),
        compiler_params=pltpu.CompilerParams(use_tc_tiling_on_sc=False),
        scratch_types=[
            pltpu.VMEM((nj, CHJ), jnp.int32),
            pltpu.VMEM((CHJ, dp), F32),
            pltpu.SemaphoreType.DMA,
        ],
    )(gather_k)


def _make_scatter(np_, dp, epp):
    per_tile = epp // NS
    nj = per_tile // CHJ
    rp = np_ // NS
    mesh = plsc.VectorSubcoreMesh(core_axis_name="c", subcore_axis_name="s")

    def scatter_k(msg_hbm, dst_hbm, zeros_hbm, out_hbm, idx_v, rows_v, acc_sh):
        c = lax.axis_index("c")
        s = lax.axis_index("s")
        w = c * NS + s
        pltpu.sync_copy(zeros_hbm.at[pl.ds(s * rp, rp)],
                        acc_sh.at[pl.ds(s * rp, rp)])
        plsc.subcore_barrier()
        pltpu.sync_copy(dst_hbm.at[w], idx_v)

        for j in range(nj):
            pltpu.sync_copy(msg_hbm.at[pl.ds(w * per_tile + j * CHJ, CHJ)], rows_v)
            pltpu.sync_copy(rows_v, acc_sh.at[idx_v.at[j]], add=True)
        plsc.subcore_barrier()
        pltpu.sync_copy(acc_sh.at[pl.ds(s * rp, rp)],
                        out_hbm.at[pl.ds(c * np_ + s * rp, rp)])

    return functools.partial(
        pl.kernel, mesh=mesh,
        out_type=jax.ShapeDtypeStruct((2 * np_, dp), F32),
        compiler_params=pltpu.CompilerParams(use_tc_tiling_on_sc=False),
        scratch_types=[
            pltpu.VMEM((nj, CHJ), jnp.int32),
            pltpu.VMEM((CHJ, dp), F32),
            pltpu.VMEM_SHARED((np_, dp), F32),
        ],
    )(scatter_k)


def _ceil_to(a, m):
    return ((a + m - 1) // m) * m


def _dense_relu(x3, w3, b3, tb):
    two, R, K = x3.shape
    O = w3.shape[2]

    def body(x_ref, w_ref, b_ref, o_ref):
        r = jnp.dot(x_ref[0], w_ref[0], preferred_element_type=F32) + b_ref[0]
        o_ref[...] = jnp.maximum(r, 0.0)[None]

    return pl.pallas_call(
        body,
        grid=(two, R // tb),
        in_specs=[
            pl.BlockSpec((1, tb, K), lambda m, i: (m, i, 0)),
            pl.BlockSpec((1, K, O), lambda m, i: (m, 0, 0)),
            pl.BlockSpec((1, 1, O), lambda m, i: (m, 0, 0)),
        ],
        out_specs=pl.BlockSpec((1, tb, O), lambda m, i: (m, i, 0)),
        out_shape=jax.ShapeDtypeStruct((two, R, O), F32),
    )(x3, w3, b3)


def _msg(hs3, rew3, wt3, rep, sel, b23, teb, D):
    two, EP, DP = hs3.shape
    KO = D * D

    def body(hs_ref, rew_ref, wt_ref, rep_ref, sel_ref, b2_ref, o_ref):
        hs = hs_ref[0]
        T = jnp.dot(hs, wt_ref[0], preferred_element_type=F32)
        Rx = jnp.dot(rew_ref[0], rep_ref[...], preferred_element_type=F32)
        acc = jnp.dot(T * Rx, sel_ref[...], preferred_element_type=F32)
        acc = acc + jnp.dot(hs, b2_ref[0], preferred_element_type=F32)
        o_ref[...] = acc[None]

    return pl.pallas_call(
        body,
        grid=(two, EP // teb),
        in_specs=[
            pl.BlockSpec((1, teb, DP), lambda m, i: (m, i, 0)),
            pl.BlockSpec((1, teb, DP), lambda m, i: (m, i, 0)),
            pl.BlockSpec((1, DP, D * D), lambda m, i: (m, 0, 0)),
            pl.BlockSpec((DP, D * D), lambda m, i: (0, 0)),
            pl.BlockSpec((D * D, DP), lambda m, i: (0, 0)),
            pl.BlockSpec((1, DP, DP), lambda m, i: (m, 0, 0)),
        ],
        out_specs=pl.BlockSpec((1, teb, DP), lambda m, i: (m, i, 0)),
        out_shape=jax.ShapeDtypeStruct((two, EP, DP), F32),
    )(hs3, rew3, wt3, rep, sel, b23)


def _update(p3, h3, cb3, w3, b3, tb):
    two, R, DP = p3.shape

    def body(p_ref, h_ref, cb_ref, w_ref, b_ref, o_ref):
        m = jnp.maximum(p_ref[0] + cb_ref[0] + h_ref[0], 0.0)
        r = jnp.dot(m, w_ref[0], preferred_element_type=F32) + b_ref[0]
        o_ref[...] = jnp.maximum(r, 0.0)[None]

    return pl.pallas_call(
        body,
        grid=(two, R // tb),
        in_specs=[
            pl.BlockSpec((1, tb, DP), lambda m, i: (m, i, 0)),
            pl.BlockSpec((1, tb, DP), lambda m, i: (m, i, 0)),
            pl.BlockSpec((1, 1, DP), lambda m, i: (m, 0, 0)),
            pl.BlockSpec((1, DP, DP), lambda m, i: (m, 0, 0)),
            pl.BlockSpec((1, 1, DP), lambda m, i: (m, 0, 0)),
        ],
        out_specs=pl.BlockSpec((1, tb, DP), lambda m, i: (m, i, 0)),
        out_shape=jax.ShapeDtypeStruct((two, R, DP), F32),
    )(p3, h3, cb3, w3, b3)


def _epilogue(h3, x3, gid3, fc1_w, fc1_b, fc2_w, fc2_b, fc3_w, fc3_b, tb, D, NG):
    two, NPAD, DP = h3.shape
    nb = NPAD // tb

    def body(h_ref, x_ref, gid_ref, w1_ref, b1_ref, w2_ref, b2_ref, w3_ref,
             b3_ref, o_ref, sums_ref):
        m = pl.program_id(0)
        i = pl.program_id(1)

        @pl.when(i == 0)
        def _():
            sums_ref[pl.ds(m * NG, NG), :] = jnp.zeros((NG, DP), F32)

        f = h_ref[0] + x_ref[0]
        col = jax.lax.broadcasted_iota(jnp.int32, (tb, DP), 1)
        f = jnp.where(col == DP - 1, 1.0, f)
        gid = gid_ref[0]
        oh = (jax.lax.broadcasted_iota(jnp.int32, (NG, tb), 0) == gid).astype(F32)
        sums_ref[pl.ds(m * NG, NG), :] += jnp.dot(oh, f, preferred_element_type=F32)

        @pl.when((m == 1) & (i == nb - 1))
        def _():
            s0 = sums_ref[pl.ds(0, NG), :]
            s1 = sums_ref[pl.ds(NG, NG), :]
            m0 = s0[:, :D] / jnp.maximum(s0[:, DP - 1:DP], 1.0)
            m1 = s1[:, :D] / jnp.maximum(s1[:, DP - 1:DP], 1.0)
            g0 = jnp.sum(m0, axis=0, keepdims=True) / NG
            g1 = jnp.sum(m1, axis=0, keepdims=True) / NG
            comb = jnp.concatenate([g0, g0, g1, g1], axis=1)
            h1 = jnp.maximum(
                jnp.dot(comb, w1_ref[...], preferred_element_type=F32) + b1_ref[...], 0.0)
            h2 = jnp.maximum(
                jnp.dot(h1, w2_ref[...], preferred_element_type=F32) + b2_ref[...], 0.0)
            o_ref[...] = jnp.dot(h2, w3_ref[...], preferred_element_type=F32) + b3_ref[...]

    return pl.pallas_call(
        body,
        grid=(two, nb),
        in_specs=[
            pl.BlockSpec((1, tb, DP), lambda m, i: (m, i, 0)),
            pl.BlockSpec((1, tb, DP), lambda m, i: (m, i, 0)),
            pl.BlockSpec((1, 1, tb), lambda m, i: (m * nb + i, 0, 0)),
            pl.BlockSpec(fc1_w.shape, lambda m, i: (0, 0)),
            pl.BlockSpec(fc1_b.shape, lambda m, i: (0, 0)),
            pl.BlockSpec(fc2_w.shape, lambda m, i: (0, 0)),
            pl.BlockSpec(fc2_b.shape, lambda m, i: (0, 0)),
            pl.BlockSpec(fc3_w.shape, lambda m, i: (0, 0)),
            pl.BlockSpec(fc3_b.shape, lambda m, i: (0, 0)),
        ],
        out_specs=pl.BlockSpec((1, 1), lambda m, i: (0, 0)),
        out_shape=jax.ShapeDtypeStruct((1, 1), F32),
        scratch_shapes=[pltpu.VMEM((2 * NG, DP), F32)],
    )(h3, x3, gid3, fc1_w, fc1_b, fc2_w, fc2_b, fc3_w, fc3_b)


def _pad2(a, r, c):
    return jnp.pad(a, ((0, r - a.shape[0]), (0, c - a.shape[1])))


def kernel(solute_x, solute_edge_index, solute_w, solute_graph_ids,
           solvent_x, solvent_edge_index, solvent_w, solvent_graph_ids,
           su_lin0_w, su_lin0_b, su_en_w1, su_en_b1, su_en_w2, su_en_b2,
           su_conv_b, su_msg_w, su_msg_b,
           sv_lin0_w, sv_lin0_b, sv_en_w1, sv_en_b1, sv_en_w2, sv_en_b2,
           sv_conv_b, sv_msg_w, sv_msg_b,
           fc1_w, fc1_b, fc2_w, fc2_b, fc3_w, fc3_b):
    N, D = solute_x.shape
    E, DE = solute_w.shape
    NG = 256
    DP = 48
    DEP = 16
    TB = 512
    TEB = 1024
    NP = _ceil_to(N + 1, TB)
    EP = _ceil_to(E, NS * CHJ)
    NJ = EP // NS // CHJ

    x3 = jnp.stack([_pad2(solute_x, NP, DP), _pad2(solvent_x, NP, DP)])
    w3 = jnp.stack([_pad2(solute_w, EP, DEP), _pad2(solvent_w, EP, DEP)])

    lin0_w3 = jnp.stack([_pad2(su_lin0_w, DP, DP), _pad2(sv_lin0_w, DP, DP)])
    lin0_b3 = jnp.stack([_pad2(su_lin0_b[None], 1, DP), _pad2(sv_lin0_b[None], 1, DP)])
    en_w13 = jnp.stack([_pad2(su_en_w1, DEP, DP), _pad2(sv_en_w1, DEP, DP)])
    en_b13 = jnp.stack([_pad2(su_en_b1[None], 1, DP), _pad2(sv_en_b1[None], 1, DP)])

    def _wt(en_w2):
        w2 = en_w2.reshape(D, D, D)
        wt = jnp.transpose(w2, (1, 0, 2))
        wt = jnp.pad(wt, ((0, DP - D), (0, 0), (0, 0)))
        return wt.reshape(DP, D * D)

    wt3 = jnp.stack([_wt(su_en_w2), _wt(sv_en_w2)])
    KO = D * D
    rep = (jnp.arange(KO)[None, :] // D
           == jnp.arange(DP)[:, None]).astype(F32)
    sel = (jnp.arange(KO)[:, None] % D
           == jnp.arange(DP)[None, :]).astype(F32)
    b23 = jnp.stack([_pad2(su_en_b2.reshape(D, D), DP, DP),
                     _pad2(sv_en_b2.reshape(D, D), DP, DP)])
    cb3 = jnp.stack([_pad2(su_conv_b[None], 1, DP), _pad2(sv_conv_b[None], 1, DP)])
    msg_w3 = jnp.stack([_pad2(su_msg_w, DP, DP), _pad2(sv_msg_w, DP, DP)])
    msg_b3 = jnp.stack([_pad2(su_msg_b[None], 1, DP), _pad2(sv_msg_b[None], 1, DP)])

    def _pad_idx(idx, fill):
        return jnp.concatenate(
            [idx.astype(jnp.int32), jnp.full((EP - E,), fill, jnp.int32)])

    src3 = jnp.concatenate([
        _pad_idx(solute_edge_index[0], N),
        _pad_idx(solvent_edge_index[0], N) + NP]).reshape(2 * NS, NJ, CHJ)
    dst3 = jnp.concatenate([
        _pad_idx(solute_edge_index[1], N),
        _pad_idx(solvent_edge_index[1], N)]).reshape(2 * NS, NJ, CHJ)
    zeros_np = jnp.zeros((NP, DP), F32)

    def _pad_gid(g):
        return jnp.concatenate(
            [g.astype(jnp.int32), jnp.full((NP - N,), NG, jnp.int32)])

    nbg = NP // TB
    gid3 = jnp.concatenate([_pad_gid(solute_graph_ids),
                            _pad_gid(solvent_graph_ids)]).reshape(2 * nbg, 1, TB)

    h3 = _dense_relu(x3, lin0_w3, lin0_b3, TB)
    rew3 = _dense_relu(w3, en_w13, en_b13, TB)

    gather_k = _make_gather(DP, EP)
    scatter_k = _make_scatter(NP, DP, EP)
    for _ in range(6):
        hs = gather_k(h3.reshape(2 * NP, DP), src3).reshape(2, EP, DP)
        msg = _msg(hs, rew3, wt3, rep, sel, b23, TEB, D)
        p = scatter_k(msg.reshape(2 * EP, DP), dst3,
                      zeros_np).reshape(2, NP, DP)
        h3 = _update(p, h3, cb3, msg_w3, msg_b3, TB)

    return _epilogue(h3, x3, gid3, fc1_w, fc1_b[None], fc2_w, fc2_b[None],
                     fc3_w, fc3_b[None], TB, D, NG)
